# initial kernel scaffold (unmeasured)
import jax
import jax.numpy as jnp
from jax import lax
from jax.experimental import pallas as pl
from jax.experimental.pallas import tpu as pltpu

P = 4
M_BLK = 1024
K_BLK = 1024
N = 8192
N_TILE = 2048
N_TILES = N // N_TILE


def _gelu(y):
    c = 0.7978845608028654
    return 0.5 * y * (1.0 + jnp.tanh(c * (y + 0.044715 * y * y * y)))


def kernel(x, w_mat):
    def body(x_ref, w_ref, out_ref, blocks, wbuf, acc,
             send_sems, recv_sems, local_sem, w_sems, out_sems):
        my = lax.axis_index("i")

        barrier = pltpu.get_barrier_semaphore()
        for d in range(1, P):
            pl.semaphore_signal(
                barrier, inc=1,
                device_id=((my + d) % P,),
                device_id_type=pl.DeviceIdType.MESH,
            )
        pl.semaphore_wait(barrier, P - 1)

        local = pltpu.make_async_copy(
            x_ref.at[pl.ds(my * M_BLK, M_BLK), :], blocks.at[0], local_sem)
        local.start()

        rdmas = []
        for d in range(1, P):
            tgt = (my + d) % P
            r = pltpu.make_async_remote_copy(
                src_ref=x_ref.at[pl.ds(tgt * M_BLK, M_BLK), :],
                dst_ref=blocks.at[d],
                send_sem=send_sems.at[d - 1],
                recv_sem=recv_sems.at[d - 1],
                device_id=(tgt,),
                device_id_type=pl.DeviceIdType.MESH,
            )
            r.start()
            rdmas.append(r)

        local.wait()
        for r in rdmas:
            r.wait()

        steps = [(t, d) for t in range(N_TILES) for d in range(P)]

        def w_dma(s):
            t, d = steps[s]
            o = (my - d + P) % P
            return pltpu.make_async_copy(
                w_ref.at[pl.ds(o * K_BLK, K_BLK), pl.ds(t * N_TILE, N_TILE)],
                wbuf.at[s % 2],
                w_sems.at[s % 2],
            )

        def out_dma(t):
            return pltpu.make_async_copy(
                acc.at[t % 2],
                out_ref.at[:, pl.ds(t * N_TILE, N_TILE)],
                out_sems.at[t % 2],
            )

        w_dma(0).start()
        for s, (t, d) in enumerate(steps):
            if s + 1 < len(steps):
                w_dma(s + 1).start()
            w_dma(s).wait()
            part = jnp.dot(blocks[d], wbuf[s % 2],
                           preferred_element_type=jnp.float32)
            slot = t % 2
            if d == 0:
                if t >= 2:
                    out_dma(t - 2).wait()
                acc[slot, :, :] = part
            else:
                acc[slot, :, :] = acc[slot, :, :] + part
            if d == P - 1:
                acc[slot, :, :] = _gelu(acc[slot, :, :])
                out_dma(t).start()

        out_dma(N_TILES - 2).wait()
        out_dma(N_TILES - 1).wait()

    return pl.pallas_call(
        body,
        out_shape=jax.ShapeDtypeStruct((M_BLK, N), jnp.float32),
        in_specs=[
            pl.BlockSpec(memory_space=pltpu.ANY),
            pl.BlockSpec(memory_space=pltpu.ANY),
        ],
        out_specs=pl.BlockSpec(memory_space=pltpu.ANY),
        scratch_shapes=[
            pltpu.VMEM((P, M_BLK, K_BLK), jnp.float32),
            pltpu.VMEM((2, K_BLK, N_TILE), jnp.float32),
            pltpu.VMEM((2, M_BLK, N_TILE), jnp.float32),
            pltpu.SemaphoreType.DMA((P - 1,)),
            pltpu.SemaphoreType.DMA((P - 1,)),
            pltpu.SemaphoreType.DMA,
            pltpu.SemaphoreType.DMA((2,)),
            pltpu.SemaphoreType.DMA((2,)),
        ],
        compiler_params=pltpu.CompilerParams(collective_id=0),
    )(x, w_mat)


# baseline (device time: 211355 ns/iter reference)
import jax
import jax.numpy as jnp
from jax import lax
from jax.experimental import pallas as pl
from jax.experimental.pallas import tpu as pltpu

P = 4
M_BLK = 1024
K_BLK = 1024
N = 8192
N_TILE = 2048
N_TILES = N // N_TILE


def _gelu(y):
    c = 0.7978845608028654
    return 0.5 * y * (1.0 + jnp.tanh(c * (y + 0.044715 * y * y * y)))


def kernel(x, w_mat):
    def body(x_ref, w_ref, out_ref, blocks, wbuf, acc,
             send_sems, recv_sems, local_sem, w_sems, out_sems):
        my = lax.axis_index("i")

        barrier = pltpu.get_barrier_semaphore()
        for d in range(1, P):
            pl.semaphore_signal(
                barrier, inc=1,
                device_id=((my + d) % P,),
                device_id_type=pl.DeviceIdType.MESH,
            )
        pl.semaphore_wait(barrier, P - 1)

        local = pltpu.make_async_copy(
            x_ref.at[pl.ds(my * M_BLK, M_BLK), :], blocks.at[0], local_sem)
        local.start()

        rdmas = []
        for d in range(1, P):
            tgt = (my + d) % P
            r = pltpu.make_async_remote_copy(
                src_ref=x_ref.at[pl.ds(tgt * M_BLK, M_BLK), :],
                dst_ref=blocks.at[d],
                send_sem=send_sems.at[d - 1],
                recv_sem=recv_sems.at[d - 1],
                device_id=(tgt,),
                device_id_type=pl.DeviceIdType.MESH,
            )
            r.start()
            rdmas.append(r)

        local.wait()
        for r in rdmas:
            r.wait()

        steps = [(t, d) for t in range(N_TILES) for d in range(P)]

        def w_dma(s):
            t, d = steps[s]
            o = (my - d + P) % P
            return pltpu.make_async_copy(
                w_ref.at[pl.ds(o * K_BLK, K_BLK), pl.ds(t * N_TILE, N_TILE)],
                wbuf.at[s % 2],
                w_sems.at[s % 2],
            )

        def out_dma(t):
            return pltpu.make_async_copy(
                acc.at[t % 2],
                out_ref.at[:, pl.ds(t * N_TILE, N_TILE)],
                out_sems.at[t % 2],
            )

        w_dma(0).start()
        for s, (t, d) in enumerate(steps):
            if s + 1 < len(steps):
                w_dma(s + 1).start()
            w_dma(s).wait()
            part = jnp.dot(blocks[d], wbuf[s % 2],
                           preferred_element_type=jnp.float32)
            slot = t % 2
            if d == 0:
                if t >= 2:
                    out_dma(t - 2).wait()
                acc[slot, :, :] = part
            else:
                acc[slot, :, :] = acc[slot, :, :] + part
            if d == P - 1:
                acc[slot, :, :] = _gelu(acc[slot, :, :])
                out_dma(t).start()

        out_dma(N_TILES - 2).wait()
        out_dma(N_TILES - 1).wait()

    return pl.pallas_call(
        body,
        out_shape=jax.ShapeDtypeStruct((M_BLK, N), jnp.float32),
        in_specs=[
            pl.BlockSpec(memory_space=pl.ANY),
            pl.BlockSpec(memory_space=pl.ANY),
        ],
        out_specs=pl.BlockSpec(memory_space=pl.ANY),
        scratch_shapes=[
            pltpu.VMEM((P, M_BLK, K_BLK), jnp.float32),
            pltpu.VMEM((2, K_BLK, N_TILE), jnp.float32),
            pltpu.VMEM((2, M_BLK, N_TILE), jnp.float32),
            pltpu.SemaphoreType.DMA((P - 1,)),
            pltpu.SemaphoreType.DMA((P - 1,)),
            pltpu.SemaphoreType.DMA,
            pltpu.SemaphoreType.DMA((2,)),
            pltpu.SemaphoreType.DMA((2,)),
        ],
        compiler_params=pltpu.CompilerParams(
            collective_id=0,
            vmem_limit_bytes=64 * 1024 * 1024,
        ),
    )(x, w_mat)


# device time: 132990 ns/iter; 1.5893x vs baseline; 1.5893x over previous
import jax
import jax.numpy as jnp
from jax import lax
from jax.experimental import pallas as pl
from jax.experimental.pallas import tpu as pltpu

P = 4
M_BLK = 1024
K_BLK = 1024
N = 8192
N_TILE = 1024
N_TILES = N // N_TILE

DS_ORDER = [0, 1, 3, 2]


def _gelu(y):
    c = 0.7978845608028654
    return 0.5 * y * (1.0 + jnp.tanh(c * (y + 0.044715 * y * y * y)))


def kernel(x, w_mat):
    x_bf = x.astype(jnp.bfloat16)

    def body(x_ref, w_ref, out_ref, blocks, bf32, wbuf, acc,
             send_sems, recv_sems, local_sem, w_sems, out_sems):
        my = lax.axis_index("i")

        barrier = pltpu.get_barrier_semaphore()
        for d in range(1, P):
            pl.semaphore_signal(
                barrier, inc=1,
                device_id=((my + d) % P,),
                device_id_type=pl.DeviceIdType.MESH,
            )
        pl.semaphore_wait(barrier, P - 1)

        rdmas = {}
        for d in range(1, P):
            tgt = (my + d) % P
            r = pltpu.make_async_remote_copy(
                src_ref=x_ref.at[pl.ds(tgt * M_BLK, M_BLK), :],
                dst_ref=blocks.at[d],
                send_sem=send_sems.at[d - 1],
                recv_sem=recv_sems.at[d - 1],
                device_id=(tgt,),
                device_id_type=pl.DeviceIdType.MESH,
            )
            r.start()
            rdmas[d] = r

        local = pltpu.make_async_copy(
            x_ref.at[pl.ds(my * M_BLK, M_BLK), :], blocks.at[0], local_sem)
        local.start()

        steps = [(d, t) for d in DS_ORDER for t in range(N_TILES)]

        def w_dma(s):
            d, t = steps[s]
            o = (my - d + P) % P
            return pltpu.make_async_copy(
                w_ref.at[pl.ds(o * K_BLK, K_BLK), pl.ds(t * N_TILE, N_TILE)],
                wbuf.at[s % 2],
                w_sems.at[s % 2],
            )

        def out_dma(t):
            nsl = pl.ds(t * N_TILE, N_TILE)
            return pltpu.make_async_copy(
                acc.at[:, nsl], out_ref.at[:, nsl], out_sems.at[t])

        w_dma(0).start()
        for s, (d, t) in enumerate(steps):
            if s + 1 < len(steps):
                w_dma(s + 1).start()
            if t == 0:
                if d == 0:
                    local.wait()
                else:
                    rdmas[d].wait_recv()
                bf32[:, :] = blocks[d].astype(jnp.float32)
            w_dma(s).wait()
            part = jnp.dot(bf32[:, :], wbuf[s % 2],
                           preferred_element_type=jnp.float32)
            nsl = slice(t * N_TILE, (t + 1) * N_TILE)
            p = s // N_TILES
            if p == 0:
                acc[:, nsl] = part
            elif p < P - 1:
                acc[:, nsl] = acc[:, nsl] + part
            else:
                acc[:, nsl] = _gelu(acc[:, nsl] + part)
                out_dma(t).start()

        for d in range(1, P):
            rdmas[d].wait_send()
        for t in range(N_TILES):
            out_dma(t).wait()

    return pl.pallas_call(
        body,
        out_shape=jax.ShapeDtypeStruct((M_BLK, N), jnp.float32),
        in_specs=[
            pl.BlockSpec(memory_space=pl.ANY),
            pl.BlockSpec(memory_space=pl.ANY),
        ],
        out_specs=pl.BlockSpec(memory_space=pl.ANY),
        scratch_shapes=[
            pltpu.VMEM((P, M_BLK, K_BLK), jnp.bfloat16),
            pltpu.VMEM((M_BLK, K_BLK), jnp.float32),
            pltpu.VMEM((2, K_BLK, N_TILE), jnp.float32),
            pltpu.VMEM((M_BLK, N), jnp.float32),
            pltpu.SemaphoreType.DMA((P - 1,)),
            pltpu.SemaphoreType.DMA((P - 1,)),
            pltpu.SemaphoreType.DMA,
            pltpu.SemaphoreType.DMA((2,)),
            pltpu.SemaphoreType.DMA((N_TILES,)),
        ],
        compiler_params=pltpu.CompilerParams(
            collective_id=0,
            vmem_limit_bytes=64 * 1024 * 1024,
        ),
    )(x_bf, w_mat)
